# Initial kernel scaffold; baseline (speedup 1.0000x reference)
import jax
import jax.numpy as jnp
from jax.experimental import pallas as pl


def kernel(kb_ids_seq, key_emb_table):
    raise NotImplementedError("write your pallas kernel here")


# speedup vs baseline: 1.4293x; 1.4293x over previous
import functools

import jax
import jax.numpy as jnp
import numpy as np
from jax import lax
from jax.experimental import pallas as pl
from jax.experimental.pallas import tpu as pltpu
from jax.experimental.pallas import tpu_sc as plsc

N_ELEMENTS = 1000000
DIM = 32
MAX_LEN = 200
B = 4096
L = 200

NC = 2
NS = 16
NW = NC * NS

TOTAL = B * L
ROWS_PER_W = TOTAL // NW
SEQS_PER_CHUNK = 8
CHUNK = SEQS_PER_CHUNK * L
NCHUNKS = ROWS_PER_W // CHUNK


def _sinusoidal_pe():
    pos = np.arange(MAX_LEN, dtype=np.float32)[:, None]
    div = np.exp(np.arange(0, DIM, 2, dtype=np.float32) * (-np.log(10000.0) / DIM))
    pe = np.zeros((MAX_LEN, DIM), dtype=np.float32)
    pe[:, 0::2] = np.sin(pos * div)
    pe[:, 1::2] = np.cos(pos * div)
    return pe


_PE = _sinusoidal_pe()


def _sc_body(table_hbm, idx_hbm, pe_hbm, out_hbm, idx_v, pe_v, rows_v, sem):
    wid = lax.axis_index("s") * NC + lax.axis_index("c")
    base = wid * ROWS_PER_W

    pltpu.sync_copy(pe_hbm, pe_v)

    def chunk_body(g, carry):
        start = base + g * CHUNK
        pltpu.sync_copy(idx_hbm.at[pl.ds(start, CHUNK)], idx_v)
        pltpu.async_copy(table_hbm.at[idx_v], rows_v, sem).wait()

        def pe_body(j, c):
            p0 = pe_v[j, pl.ds(0, 16)]
            p1 = pe_v[j, pl.ds(16, 16)]
            for s in range(SEQS_PER_CHUNK):
                r = s * L + j
                rows_v[r, pl.ds(0, 16)] = rows_v[r, pl.ds(0, 16)] + p0
                rows_v[r, pl.ds(16, 16)] = rows_v[r, pl.ds(16, 16)] + p1
            return c

        lax.fori_loop(0, L, pe_body, 0, unroll=False)

        pltpu.sync_copy(rows_v, out_hbm.at[pl.ds(start, CHUNK)])
        return carry

    lax.fori_loop(0, NCHUNKS, chunk_body, 0, unroll=False)


@jax.jit
def _run(idx_flat, table):
    mesh = plsc.VectorSubcoreMesh(core_axis_name="c", subcore_axis_name="s")
    f = pl.kernel(
        _sc_body,
        out_type=jax.ShapeDtypeStruct((TOTAL, DIM), jnp.float32),
        mesh=mesh,
        scratch_types=[
            pltpu.VMEM((CHUNK,), jnp.int32),
            pltpu.VMEM((MAX_LEN, DIM), jnp.float32),
            pltpu.VMEM((CHUNK, DIM), jnp.float32),
            pltpu.SemaphoreType.DMA,
        ],
        compiler_params=pltpu.CompilerParams(use_tc_tiling_on_sc=False),
    )
    return f(table, idx_flat, jnp.asarray(_PE))


def kernel(kb_ids_seq, key_emb_table):
    out = _run(kb_ids_seq.reshape(TOTAL), key_emb_table)
    return out.reshape(B, L, DIM)
